# baseline (speedup 1.0000x reference)
import functools

import jax
import jax.numpy as jnp
from jax import lax
from jax.experimental import pallas as pl
from jax.experimental.pallas import tpu as pltpu
from jax.experimental.pallas import tpu_sc as plsc

N = 10000
D = 128
N_PAIRS = 160000

NW = 32
CH = 128
B_PAD = 163840
NC0 = 80
NC1 = 0
NBUF = 2


def _x_body(emb_ref, w_ref, x_ref):
    x_ref[...] = jnp.dot(emb_ref[...], w_ref[...],
                         preferred_element_type=jnp.float32)


def _compute_x(emb, w0):
    return pl.pallas_call(
        _x_body,
        grid=(10,),
        in_specs=[pl.BlockSpec((1000, D), lambda i: (i, 0)),
                  pl.BlockSpec((D, D), lambda i: (0, 0))],
        out_specs=pl.BlockSpec((1000, D), lambda i: (i, 0)),
        out_shape=jax.ShapeDtypeStruct((N, D), jnp.float32),
    )(emb, w0)


def _h_body(a_ref, x_ref, h_ref):
    t = jnp.tanh(jnp.dot(a_ref[...], x_ref[...],
                         preferred_element_type=jnp.float32))
    lo = jax.lax.bitcast_convert_type(
        t[:, :D // 2].astype(jnp.bfloat16), jnp.uint16).astype(jnp.uint32)
    hi = jax.lax.bitcast_convert_type(
        t[:, D // 2:].astype(jnp.bfloat16), jnp.uint16).astype(jnp.uint32)
    h_ref[...] = jax.lax.bitcast_convert_type(lo | (hi << 16), jnp.float32)


def _compute_h(a, x):
    bm = 400
    return pl.pallas_call(
        _h_body,
        grid=(N // bm,),
        in_specs=[pl.BlockSpec((bm, N), lambda i: (i, 0)),
                  pl.BlockSpec((N, D), lambda i: (0, 0))],
        out_specs=pl.BlockSpec((bm, D // 2), lambda i: (i, 0)),
        out_shape=jax.ShapeDtypeStruct((N, D // 2), jnp.float32),
    )(a, x)


def _dist_body(src_hbm, dst_hbm, h_hbm, out_hbm,
               sidx, didx,
               srows0, drows0, srows1, drows1, srows2, drows2, srows3, drows3,
               dch, tbuf,
               sem_s0, sem_d0, sem_s1, sem_d1,
               sem_s2, sem_d2, sem_s3, sem_d3):
    c = lax.axis_index("c")
    s = lax.axis_index("s")

    sets = ((srows0, drows0, sem_s0, sem_d0),
            (srows1, drows1, sem_s1, sem_d1),
            (srows2, drows2, sem_s2, sem_d2),
            (srows3, drows3, sem_s3, sem_d3))

    def start(ci, st):
        srows, drows, sem_s, sem_d = st
        cp_s = pltpu.async_copy(h_hbm.at[sidx.at[ci]], srows, sem_s)
        cp_d = pltpu.async_copy(h_hbm.at[didx.at[ci]], drows, sem_d)
        return cp_s, cp_d

    def wait(st):
        srows, drows, sem_s, sem_d = st
        pltpu.make_async_copy(h_hbm.at[sidx.at[0]], srows, sem_s).wait()
        pltpu.make_async_copy(h_hbm.at[didx.at[0]], drows, sem_d).wait()

    def compute(ci, srows, drows):
        def block_body(b, carry):
            for p in range(16):
                row = b * 16 + p
                acc = jnp.zeros((16,), jnp.float32)
                for k in range(D // 32):
                    sv = srows[row, pl.ds(k * 16, 16)]
                    tv = drows[row, pl.ds(k * 16, 16)]
                    s32 = plsc.bitcast(sv, jnp.bfloat16)
                    t32 = plsc.bitcast(tv, jnp.bfloat16)
                    sa, sb = plsc.unpack(s32, format=plsc.PackFormat.INTERLEAVED)
                    ta, tb = plsc.unpack(t32, format=plsc.PackFormat.INTERLEAVED)
                    dfa = sa - ta
                    dfb = sb - tb
                    acc = acc + dfa * dfa
                    acc = acc + dfb * dfb
                tbuf[pl.ds(p * 17, 16)] = acc
            tot = jnp.zeros((16,), jnp.float32)
            lanes = lax.iota(jnp.int32, 16) * 17
            for l in range(16):
                tot = tot + plsc.load_gather(tbuf, [lanes + l])
            dch[pl.ds(ci * CH + b * 16, 16)] = tot * (1.0 / D)
            return carry

        lax.fori_loop(0, CH // 16, block_body, 0)

    def run(base_chunk, n_chunks):
        pltpu.sync_copy(src_hbm.at[pl.ds(base_chunk, n_chunks)],
                        sidx.at[pl.ds(0, n_chunks)])
        pltpu.sync_copy(dst_hbm.at[pl.ds(base_chunk, n_chunks)],
                        didx.at[pl.ds(0, n_chunks)])

        for j in range(NBUF - 1):
            start(j, sets[j])

        def ring_body(i, carry):
            base = NBUF * i
            start(base + NBUF - 1, sets[NBUF - 1])
            for j in range(NBUF):
                wait(sets[j])
                compute(base + j, *sets[j][:2])
                if j < NBUF - 1:
                    start(jnp.minimum(base + NBUF + j, n_chunks - 1), sets[j])
            return carry

        lax.fori_loop(0, n_chunks // NBUF, ring_body, 0)

        for j in range(NBUF - 1):
            wait(sets[j])

        pltpu.sync_copy(dch.at[pl.ds(0, n_chunks * CH)],
                        out_hbm.at[pl.ds(base_chunk * CH, n_chunks * CH)])

    @pl.when(c == 0)
    def _():
        run(s * NC0, NC0)

    if NC1 > 0:
        @pl.when(c == 1)
        def _():
            run(16 * NC0 + s * NC1, NC1)


def _compute_dist(src_idx2d, dst_idx2d, h):
    mesh = plsc.VectorSubcoreMesh(core_axis_name="c", subcore_axis_name="s")
    f = functools.partial(
        pl.kernel,
        out_type=jax.ShapeDtypeStruct((B_PAD,), jnp.float32),
        mesh=mesh,
        scratch_types=[
            pltpu.VMEM((NC0, CH), jnp.int32),
            pltpu.VMEM((NC0, CH), jnp.int32),
            pltpu.VMEM((CH, D // 2), jnp.float32),
            pltpu.VMEM((CH, D // 2), jnp.float32),
            pltpu.VMEM((CH, D // 2), jnp.float32),
            pltpu.VMEM((CH, D // 2), jnp.float32),
            pltpu.VMEM((CH, D // 2), jnp.float32),
            pltpu.VMEM((CH, D // 2), jnp.float32),
            pltpu.VMEM((CH, D // 2), jnp.float32),
            pltpu.VMEM((CH, D // 2), jnp.float32),
            pltpu.VMEM((NC0 * CH,), jnp.float32),
            pltpu.VMEM((16 * 17,), jnp.float32),
            pltpu.SemaphoreType.DMA,
            pltpu.SemaphoreType.DMA,
            pltpu.SemaphoreType.DMA,
            pltpu.SemaphoreType.DMA,
            pltpu.SemaphoreType.DMA,
            pltpu.SemaphoreType.DMA,
            pltpu.SemaphoreType.DMA,
            pltpu.SemaphoreType.DMA,
        ],
        compiler_params=pltpu.CompilerParams(needs_layout_passes=False,
                                             use_tc_tiling_on_sc=False),
    )(_dist_body)
    return f(src_idx2d, dst_idx2d, h)


def _loss_body(d_ref, l_ref, o_ref):
    dist = d_ref[...]
    lab = l_ref[...].astype(jnp.float32)
    p = jnp.clip(jnp.exp(-dist), 1e-7, 1.0 - 1e-7)
    t = lab * jnp.log(p) + (1.0 - lab) * jnp.log(1.0 - p)
    o_ref[...] = jnp.full((1, 1), -jnp.mean(t), jnp.float32)


def _compute_loss(dist, labels):
    d2 = dist[:N_PAIRS].reshape(1250, 128)
    l2 = labels.reshape(1250, 128)
    out = pl.pallas_call(
        _loss_body,
        out_shape=jax.ShapeDtypeStruct((1, 1), jnp.float32),
    )(d2, l2)
    return out[0, 0]


def kernel(pairs, labels, A, emb, W):
    src = pairs[:, 0]
    dst = pairs[:, 1]
    pad = jnp.zeros((B_PAD - N_PAIRS,), jnp.int32)
    src_p = jnp.concatenate([src, pad]).reshape(B_PAD // CH, CH)
    dst_p = jnp.concatenate([dst, pad]).reshape(B_PAD // CH, CH)

    x = _compute_x(emb, W[0])
    h_packed = _compute_h(A, x)
    dist = _compute_dist(src_p, dst_p, h_packed)
    return _compute_loss(dist, labels)


# speedup vs baseline: 1.3031x; 1.3031x over previous
import functools

import jax
import jax.numpy as jnp
from jax import lax
from jax.experimental import pallas as pl
from jax.experimental.pallas import tpu as pltpu
from jax.experimental.pallas import tpu_sc as plsc

N = 10000
D = 128
N_PAIRS = 160000

NW = 32
CH = 128
B_PAD = 163840
NC0 = 64
NC1 = 16
NBUF = 2


def _x_body(emb_ref, w_ref, x_ref):
    x_ref[...] = jnp.dot(emb_ref[...], w_ref[...],
                         preferred_element_type=jnp.float32)


def _compute_x(emb, w0):
    return pl.pallas_call(
        _x_body,
        grid=(10,),
        in_specs=[pl.BlockSpec((1000, D), lambda i: (i, 0)),
                  pl.BlockSpec((D, D), lambda i: (0, 0))],
        out_specs=pl.BlockSpec((1000, D), lambda i: (i, 0)),
        out_shape=jax.ShapeDtypeStruct((N, D), jnp.float32),
    )(emb, w0)


def _h_body(a_ref, x_ref, h_ref):
    t = jnp.tanh(jnp.dot(a_ref[...], x_ref[...],
                         preferred_element_type=jnp.float32))
    lo = jax.lax.bitcast_convert_type(
        t[:, :D // 2].astype(jnp.bfloat16), jnp.uint16).astype(jnp.uint32)
    hi = jax.lax.bitcast_convert_type(
        t[:, D // 2:].astype(jnp.bfloat16), jnp.uint16).astype(jnp.uint32)
    h_ref[...] = jax.lax.bitcast_convert_type(lo | (hi << 16), jnp.float32)


def _compute_h(a, x):
    bm = 400
    return pl.pallas_call(
        _h_body,
        grid=(N // bm,),
        in_specs=[pl.BlockSpec((bm, N), lambda i: (i, 0)),
                  pl.BlockSpec((N, D), lambda i: (0, 0))],
        out_specs=pl.BlockSpec((bm, D // 2), lambda i: (i, 0)),
        out_shape=jax.ShapeDtypeStruct((N, D // 2), jnp.float32),
    )(a, x)


def _dist_body(src_hbm, dst_hbm, h_hbm, out_hbm,
               sidx, didx,
               srows0, drows0, srows1, drows1, srows2, drows2, srows3, drows3,
               dch, tbuf,
               sem_s0, sem_d0, sem_s1, sem_d1,
               sem_s2, sem_d2, sem_s3, sem_d3):
    c = lax.axis_index("c")
    s = lax.axis_index("s")

    sets = ((srows0, drows0, sem_s0, sem_d0),
            (srows1, drows1, sem_s1, sem_d1),
            (srows2, drows2, sem_s2, sem_d2),
            (srows3, drows3, sem_s3, sem_d3))

    def start(ci, st):
        srows, drows, sem_s, sem_d = st
        cp_s = pltpu.async_copy(h_hbm.at[sidx.at[ci]], srows, sem_s)
        cp_d = pltpu.async_copy(h_hbm.at[didx.at[ci]], drows, sem_d)
        return cp_s, cp_d

    def wait(st):
        srows, drows, sem_s, sem_d = st
        pltpu.make_async_copy(h_hbm.at[sidx.at[0]], srows, sem_s).wait()
        pltpu.make_async_copy(h_hbm.at[didx.at[0]], drows, sem_d).wait()

    def compute(ci, srows, drows):
        def block_body(b, carry):
            for p in range(16):
                row = b * 16 + p
                acc = jnp.zeros((16,), jnp.float32)
                for k in range(D // 32):
                    sv = srows[row, pl.ds(k * 16, 16)]
                    tv = drows[row, pl.ds(k * 16, 16)]
                    s32 = plsc.bitcast(sv, jnp.bfloat16)
                    t32 = plsc.bitcast(tv, jnp.bfloat16)
                    sa, sb = plsc.unpack(s32, format=plsc.PackFormat.INTERLEAVED)
                    ta, tb = plsc.unpack(t32, format=plsc.PackFormat.INTERLEAVED)
                    dfa = sa - ta
                    dfb = sb - tb
                    acc = acc + dfa * dfa
                    acc = acc + dfb * dfb
                tbuf[pl.ds(p * 17, 16)] = acc
            tot = jnp.zeros((16,), jnp.float32)
            lanes = lax.iota(jnp.int32, 16) * 17
            for l in range(16):
                tot = tot + plsc.load_gather(tbuf, [lanes + l])
            dch[pl.ds(ci * CH + b * 16, 16)] = tot * (1.0 / D)
            return carry

        lax.fori_loop(0, CH // 16, block_body, 0)

    def run(base_chunk, n_chunks):
        pltpu.sync_copy(src_hbm.at[pl.ds(base_chunk, n_chunks)],
                        sidx.at[pl.ds(0, n_chunks)])
        pltpu.sync_copy(dst_hbm.at[pl.ds(base_chunk, n_chunks)],
                        didx.at[pl.ds(0, n_chunks)])

        for j in range(NBUF - 1):
            start(j, sets[j])

        def ring_body(i, carry):
            base = NBUF * i
            start(base + NBUF - 1, sets[NBUF - 1])
            for j in range(NBUF):
                wait(sets[j])
                compute(base + j, *sets[j][:2])
                if j < NBUF - 1:
                    start(jnp.minimum(base + NBUF + j, n_chunks - 1), sets[j])
            return carry

        lax.fori_loop(0, n_chunks // NBUF, ring_body, 0)

        for j in range(NBUF - 1):
            wait(sets[j])

        pltpu.sync_copy(dch.at[pl.ds(0, n_chunks * CH)],
                        out_hbm.at[pl.ds(base_chunk * CH, n_chunks * CH)])

    @pl.when(c == 0)
    def _():
        run(s * NC0, NC0)

    if NC1 > 0:
        @pl.when(c == 1)
        def _():
            run(16 * NC0 + s * NC1, NC1)


def _compute_dist(src_idx2d, dst_idx2d, h):
    mesh = plsc.VectorSubcoreMesh(core_axis_name="c", subcore_axis_name="s")
    f = functools.partial(
        pl.kernel,
        out_type=jax.ShapeDtypeStruct((B_PAD,), jnp.float32),
        mesh=mesh,
        scratch_types=[
            pltpu.VMEM((NC0, CH), jnp.int32),
            pltpu.VMEM((NC0, CH), jnp.int32),
            pltpu.VMEM((CH, D // 2), jnp.float32),
            pltpu.VMEM((CH, D // 2), jnp.float32),
            pltpu.VMEM((CH, D // 2), jnp.float32),
            pltpu.VMEM((CH, D // 2), jnp.float32),
            pltpu.VMEM((CH, D // 2), jnp.float32),
            pltpu.VMEM((CH, D // 2), jnp.float32),
            pltpu.VMEM((CH, D // 2), jnp.float32),
            pltpu.VMEM((CH, D // 2), jnp.float32),
            pltpu.VMEM((NC0 * CH,), jnp.float32),
            pltpu.VMEM((16 * 17,), jnp.float32),
            pltpu.SemaphoreType.DMA,
            pltpu.SemaphoreType.DMA,
            pltpu.SemaphoreType.DMA,
            pltpu.SemaphoreType.DMA,
            pltpu.SemaphoreType.DMA,
            pltpu.SemaphoreType.DMA,
            pltpu.SemaphoreType.DMA,
            pltpu.SemaphoreType.DMA,
        ],
        compiler_params=pltpu.CompilerParams(needs_layout_passes=False,
                                             use_tc_tiling_on_sc=False),
    )(_dist_body)
    return f(src_idx2d, dst_idx2d, h)


def _loss_body(d_ref, l_ref, o_ref):
    dist = d_ref[...]
    lab = l_ref[...].astype(jnp.float32)
    p = jnp.clip(jnp.exp(-dist), 1e-7, 1.0 - 1e-7)
    t = lab * jnp.log(p) + (1.0 - lab) * jnp.log(1.0 - p)
    o_ref[...] = jnp.full((1, 1), -jnp.mean(t), jnp.float32)


def _compute_loss(dist, labels):
    d2 = dist[:N_PAIRS].reshape(1250, 128)
    l2 = labels.reshape(1250, 128)
    out = pl.pallas_call(
        _loss_body,
        out_shape=jax.ShapeDtypeStruct((1, 1), jnp.float32),
    )(d2, l2)
    return out[0, 0]


def kernel(pairs, labels, A, emb, W):
    src = pairs[:, 0]
    dst = pairs[:, 1]
    pad = jnp.zeros((B_PAD - N_PAIRS,), jnp.int32)
    src_p = jnp.concatenate([src, pad]).reshape(B_PAD // CH, CH)
    dst_p = jnp.concatenate([dst, pad]).reshape(B_PAD // CH, CH)

    x = _compute_x(emb, W[0])
    h_packed = _compute_h(A, x)
    dist = _compute_dist(src_p, dst_p, h_packed)
    return _compute_loss(dist, labels)


# speedup vs baseline: 1.3346x; 1.0242x over previous
import functools

import jax
import jax.numpy as jnp
from jax import lax
from jax.experimental import pallas as pl
from jax.experimental.pallas import tpu as pltpu
from jax.experimental.pallas import tpu_sc as plsc

N = 10000
D = 128
N_PAIRS = 160000

NW = 32
CH = 128
B_PAD = 163840
NC0 = 72
NC1 = 8
NBUF = 2


def _x_body(emb_ref, w_ref, x_ref):
    x_ref[...] = jnp.dot(emb_ref[...], w_ref[...],
                         preferred_element_type=jnp.float32)


def _compute_x(emb, w0):
    return pl.pallas_call(
        _x_body,
        grid=(10,),
        in_specs=[pl.BlockSpec((1000, D), lambda i: (i, 0)),
                  pl.BlockSpec((D, D), lambda i: (0, 0))],
        out_specs=pl.BlockSpec((1000, D), lambda i: (i, 0)),
        out_shape=jax.ShapeDtypeStruct((N, D), jnp.float32),
    )(emb, w0)


def _h_body(a_ref, x_ref, h_ref):
    t = jnp.tanh(jnp.dot(a_ref[...], x_ref[...],
                         preferred_element_type=jnp.float32))
    lo = jax.lax.bitcast_convert_type(
        t[:, :D // 2].astype(jnp.bfloat16), jnp.uint16).astype(jnp.uint32)
    hi = jax.lax.bitcast_convert_type(
        t[:, D // 2:].astype(jnp.bfloat16), jnp.uint16).astype(jnp.uint32)
    h_ref[...] = jax.lax.bitcast_convert_type(lo | (hi << 16), jnp.float32)


def _compute_h(a, x):
    bm = 400
    return pl.pallas_call(
        _h_body,
        grid=(N // bm,),
        in_specs=[pl.BlockSpec((bm, N), lambda i: (i, 0)),
                  pl.BlockSpec((N, D), lambda i: (0, 0))],
        out_specs=pl.BlockSpec((bm, D // 2), lambda i: (i, 0)),
        out_shape=jax.ShapeDtypeStruct((N, D // 2), jnp.float32),
    )(a, x)


def _dist_body(src_hbm, dst_hbm, h_hbm, out_hbm,
               sidx, didx,
               srows0, drows0, srows1, drows1, srows2, drows2, srows3, drows3,
               dch, tbuf,
               sem_s0, sem_d0, sem_s1, sem_d1,
               sem_s2, sem_d2, sem_s3, sem_d3):
    c = lax.axis_index("c")
    s = lax.axis_index("s")

    sets = ((srows0, drows0, sem_s0, sem_d0),
            (srows1, drows1, sem_s1, sem_d1),
            (srows2, drows2, sem_s2, sem_d2),
            (srows3, drows3, sem_s3, sem_d3))

    def start(ci, st):
        srows, drows, sem_s, sem_d = st
        cp_s = pltpu.async_copy(h_hbm.at[sidx.at[ci]], srows, sem_s)
        cp_d = pltpu.async_copy(h_hbm.at[didx.at[ci]], drows, sem_d)
        return cp_s, cp_d

    def wait(st):
        srows, drows, sem_s, sem_d = st
        pltpu.make_async_copy(h_hbm.at[sidx.at[0]], srows, sem_s).wait()
        pltpu.make_async_copy(h_hbm.at[didx.at[0]], drows, sem_d).wait()

    def compute(ci, srows, drows):
        def block_body(b, carry):
            for p in range(16):
                row = b * 16 + p
                acc = jnp.zeros((16,), jnp.float32)
                for k in range(D // 32):
                    sv = srows[row, pl.ds(k * 16, 16)]
                    tv = drows[row, pl.ds(k * 16, 16)]
                    s32 = plsc.bitcast(sv, jnp.bfloat16)
                    t32 = plsc.bitcast(tv, jnp.bfloat16)
                    sa, sb = plsc.unpack(s32, format=plsc.PackFormat.INTERLEAVED)
                    ta, tb = plsc.unpack(t32, format=plsc.PackFormat.INTERLEAVED)
                    dfa = sa - ta
                    dfb = sb - tb
                    acc = acc + dfa * dfa
                    acc = acc + dfb * dfb
                tbuf[pl.ds(p * 17, 16)] = acc
            tot = jnp.zeros((16,), jnp.float32)
            lanes = lax.iota(jnp.int32, 16) * 17
            for l in range(16):
                tot = tot + plsc.load_gather(tbuf, [lanes + l])
            dch[pl.ds(ci * CH + b * 16, 16)] = tot * (1.0 / D)
            return carry

        lax.fori_loop(0, CH // 16, block_body, 0)

    def run(base_chunk, n_chunks):
        pltpu.sync_copy(src_hbm.at[pl.ds(base_chunk, n_chunks)],
                        sidx.at[pl.ds(0, n_chunks)])
        pltpu.sync_copy(dst_hbm.at[pl.ds(base_chunk, n_chunks)],
                        didx.at[pl.ds(0, n_chunks)])

        for j in range(NBUF - 1):
            start(j, sets[j])

        def ring_body(i, carry):
            base = NBUF * i
            start(base + NBUF - 1, sets[NBUF - 1])
            for j in range(NBUF):
                wait(sets[j])
                compute(base + j, *sets[j][:2])
                if j < NBUF - 1:
                    start(jnp.minimum(base + NBUF + j, n_chunks - 1), sets[j])
            return carry

        lax.fori_loop(0, n_chunks // NBUF, ring_body, 0)

        for j in range(NBUF - 1):
            wait(sets[j])

        pltpu.sync_copy(dch.at[pl.ds(0, n_chunks * CH)],
                        out_hbm.at[pl.ds(base_chunk * CH, n_chunks * CH)])

    @pl.when(c == 0)
    def _():
        run(s * NC0, NC0)

    if NC1 > 0:
        @pl.when(c == 1)
        def _():
            run(16 * NC0 + s * NC1, NC1)


def _compute_dist(src_idx2d, dst_idx2d, h):
    mesh = plsc.VectorSubcoreMesh(core_axis_name="c", subcore_axis_name="s")
    f = functools.partial(
        pl.kernel,
        out_type=jax.ShapeDtypeStruct((B_PAD,), jnp.float32),
        mesh=mesh,
        scratch_types=[
            pltpu.VMEM((NC0, CH), jnp.int32),
            pltpu.VMEM((NC0, CH), jnp.int32),
            pltpu.VMEM((CH, D // 2), jnp.float32),
            pltpu.VMEM((CH, D // 2), jnp.float32),
            pltpu.VMEM((CH, D // 2), jnp.float32),
            pltpu.VMEM((CH, D // 2), jnp.float32),
            pltpu.VMEM((CH, D // 2), jnp.float32),
            pltpu.VMEM((CH, D // 2), jnp.float32),
            pltpu.VMEM((CH, D // 2), jnp.float32),
            pltpu.VMEM((CH, D // 2), jnp.float32),
            pltpu.VMEM((NC0 * CH,), jnp.float32),
            pltpu.VMEM((16 * 17,), jnp.float32),
            pltpu.SemaphoreType.DMA,
            pltpu.SemaphoreType.DMA,
            pltpu.SemaphoreType.DMA,
            pltpu.SemaphoreType.DMA,
            pltpu.SemaphoreType.DMA,
            pltpu.SemaphoreType.DMA,
            pltpu.SemaphoreType.DMA,
            pltpu.SemaphoreType.DMA,
        ],
        compiler_params=pltpu.CompilerParams(needs_layout_passes=False,
                                             use_tc_tiling_on_sc=False),
    )(_dist_body)
    return f(src_idx2d, dst_idx2d, h)


def _loss_body(d_ref, l_ref, o_ref):
    dist = d_ref[...]
    lab = l_ref[...].astype(jnp.float32)
    p = jnp.clip(jnp.exp(-dist), 1e-7, 1.0 - 1e-7)
    t = lab * jnp.log(p) + (1.0 - lab) * jnp.log(1.0 - p)
    o_ref[...] = jnp.full((1, 1), -jnp.mean(t), jnp.float32)


def _compute_loss(dist, labels):
    d2 = dist[:N_PAIRS].reshape(1250, 128)
    l2 = labels.reshape(1250, 128)
    out = pl.pallas_call(
        _loss_body,
        out_shape=jax.ShapeDtypeStruct((1, 1), jnp.float32),
    )(d2, l2)
    return out[0, 0]


def kernel(pairs, labels, A, emb, W):
    src = pairs[:, 0]
    dst = pairs[:, 1]
    pad = jnp.zeros((B_PAD - N_PAIRS,), jnp.int32)
    src_p = jnp.concatenate([src, pad]).reshape(B_PAD // CH, CH)
    dst_p = jnp.concatenate([dst, pad]).reshape(B_PAD // CH, CH)

    x = _compute_x(emb, W[0])
    h_packed = _compute_h(A, x)
    dist = _compute_dist(src_p, dst_p, h_packed)
    return _compute_loss(dist, labels)


# speedup vs baseline: 1.3385x; 1.0030x over previous
import functools

import jax
import jax.numpy as jnp
from jax import lax
from jax.experimental import pallas as pl
from jax.experimental.pallas import tpu as pltpu
from jax.experimental.pallas import tpu_sc as plsc

N = 10000
D = 128
N_PAIRS = 160000

NW = 32
CH = 128
B_PAD = 163840
NC0 = 72
NC1 = 8
NBUF = 2


def _x_body(emb_ref, w_ref, x_ref):
    x_ref[...] = jnp.dot(emb_ref[...], w_ref[...],
                         preferred_element_type=jnp.float32)


def _compute_x(emb, w0):
    return pl.pallas_call(
        _x_body,
        grid=(10,),
        in_specs=[pl.BlockSpec((1000, D), lambda i: (i, 0)),
                  pl.BlockSpec((D, D), lambda i: (0, 0))],
        out_specs=pl.BlockSpec((1000, D), lambda i: (i, 0)),
        out_shape=jax.ShapeDtypeStruct((N, D), jnp.float32),
    )(emb, w0)


def _h_body(a_ref, x_ref, h_ref):
    t = jnp.tanh(jnp.dot(a_ref[...], x_ref[...],
                         preferred_element_type=jnp.float32))
    lo = jax.lax.bitcast_convert_type(
        t[:, :D // 2].astype(jnp.bfloat16), jnp.uint16).astype(jnp.uint32)
    hi = jax.lax.bitcast_convert_type(
        t[:, D // 2:].astype(jnp.bfloat16), jnp.uint16).astype(jnp.uint32)
    h_ref[...] = jax.lax.bitcast_convert_type(lo | (hi << 16), jnp.float32)


def _compute_h(a, x):
    bm = 400
    return pl.pallas_call(
        _h_body,
        grid=(N // bm,),
        in_specs=[pl.BlockSpec((bm, N), lambda i: (i, 0)),
                  pl.BlockSpec((N, D), lambda i: (0, 0))],
        out_specs=pl.BlockSpec((bm, D // 2), lambda i: (i, 0)),
        out_shape=jax.ShapeDtypeStruct((N, D // 2), jnp.float32),
    )(a, x)


def _dist_body(src_hbm, dst_hbm, h_hbm, out_hbm,
               sidx, didx,
               srows0, drows0, srows1, drows1, srows2, drows2, srows3, drows3,
               dch, tbuf,
               sem_s0, sem_d0, sem_s1, sem_d1,
               sem_s2, sem_d2, sem_s3, sem_d3):
    c = lax.axis_index("c")
    s = lax.axis_index("s")

    sets = ((srows0, drows0, sem_s0, sem_d0),
            (srows1, drows1, sem_s1, sem_d1),
            (srows2, drows2, sem_s2, sem_d2),
            (srows3, drows3, sem_s3, sem_d3))

    def start(ci, st):
        srows, drows, sem_s, sem_d = st
        cp_s = pltpu.async_copy(h_hbm.at[sidx.at[ci]], srows, sem_s)
        cp_d = pltpu.async_copy(h_hbm.at[didx.at[ci]], drows, sem_d)
        return cp_s, cp_d

    def wait(st):
        srows, drows, sem_s, sem_d = st
        pltpu.make_async_copy(h_hbm.at[sidx.at[0]], srows, sem_s).wait()
        pltpu.make_async_copy(h_hbm.at[didx.at[0]], drows, sem_d).wait()

    def compute(ci, srows, drows):
        def block_body(b, carry):
            for p in range(16):
                row = b * 16 + p
                acc = jnp.zeros((16,), jnp.float32)
                for k in range(D // 32):
                    sv = srows[row, pl.ds(k * 16, 16)]
                    tv = drows[row, pl.ds(k * 16, 16)]
                    s32 = plsc.bitcast(sv, jnp.bfloat16)
                    t32 = plsc.bitcast(tv, jnp.bfloat16)
                    sa, sb = plsc.unpack(s32, format=plsc.PackFormat.INTERLEAVED)
                    ta, tb = plsc.unpack(t32, format=plsc.PackFormat.INTERLEAVED)
                    dfa = sa - ta
                    dfb = sb - tb
                    acc = acc + dfa * dfa
                    acc = acc + dfb * dfb
                tbuf[pl.ds(p * 17, 16)] = acc
            tot = jnp.zeros((16,), jnp.float32)
            lanes = lax.iota(jnp.int32, 16) * 17
            for l in range(16):
                tot = tot + plsc.load_gather(tbuf, [lanes + l])
            dch[pl.ds(ci * CH + b * 16, 16)] = tot * (1.0 / D)
            return carry

        lax.fori_loop(0, CH // 16, block_body, 0)

    def run(base_chunk, n_chunks):
        cp_i = pltpu.async_copy(src_hbm.at[pl.ds(base_chunk, n_chunks)],
                                sidx.at[pl.ds(0, n_chunks)], sem_s0)
        cp_j = pltpu.async_copy(dst_hbm.at[pl.ds(base_chunk, n_chunks)],
                                didx.at[pl.ds(0, n_chunks)], sem_d0)
        cp_i.wait()
        cp_j.wait()

        for j in range(NBUF - 1):
            start(j, sets[j])

        def ring_body(i, carry):
            base = NBUF * i
            start(base + NBUF - 1, sets[NBUF - 1])
            for j in range(NBUF):
                wait(sets[j])
                compute(base + j, *sets[j][:2])
                if j < NBUF - 1:
                    start(jnp.minimum(base + NBUF + j, n_chunks - 1), sets[j])
            return carry

        lax.fori_loop(0, n_chunks // NBUF, ring_body, 0)

        for j in range(NBUF - 1):
            wait(sets[j])

        pltpu.sync_copy(dch.at[pl.ds(0, n_chunks * CH)],
                        out_hbm.at[pl.ds(base_chunk * CH, n_chunks * CH)])

    @pl.when(c == 0)
    def _():
        run(s * NC0, NC0)

    if NC1 > 0:
        @pl.when(c == 1)
        def _():
            run(16 * NC0 + s * NC1, NC1)


def _compute_dist(src_idx2d, dst_idx2d, h):
    mesh = plsc.VectorSubcoreMesh(core_axis_name="c", subcore_axis_name="s")
    f = functools.partial(
        pl.kernel,
        out_type=jax.ShapeDtypeStruct((B_PAD,), jnp.float32),
        mesh=mesh,
        scratch_types=[
            pltpu.VMEM((NC0, CH), jnp.int32),
            pltpu.VMEM((NC0, CH), jnp.int32),
            pltpu.VMEM((CH, D // 2), jnp.float32),
            pltpu.VMEM((CH, D // 2), jnp.float32),
            pltpu.VMEM((CH, D // 2), jnp.float32),
            pltpu.VMEM((CH, D // 2), jnp.float32),
            pltpu.VMEM((CH, D // 2), jnp.float32),
            pltpu.VMEM((CH, D // 2), jnp.float32),
            pltpu.VMEM((CH, D // 2), jnp.float32),
            pltpu.VMEM((CH, D // 2), jnp.float32),
            pltpu.VMEM((NC0 * CH,), jnp.float32),
            pltpu.VMEM((16 * 17,), jnp.float32),
            pltpu.SemaphoreType.DMA,
            pltpu.SemaphoreType.DMA,
            pltpu.SemaphoreType.DMA,
            pltpu.SemaphoreType.DMA,
            pltpu.SemaphoreType.DMA,
            pltpu.SemaphoreType.DMA,
            pltpu.SemaphoreType.DMA,
            pltpu.SemaphoreType.DMA,
        ],
        compiler_params=pltpu.CompilerParams(needs_layout_passes=False,
                                             use_tc_tiling_on_sc=False),
    )(_dist_body)
    return f(src_idx2d, dst_idx2d, h)


def _loss_body(d_ref, l_ref, o_ref):
    dist = d_ref[...]
    lab = l_ref[...].astype(jnp.float32)
    p = jnp.clip(jnp.exp(-dist), 1e-7, 1.0 - 1e-7)
    t = lab * jnp.log(p) + (1.0 - lab) * jnp.log(1.0 - p)
    o_ref[...] = jnp.full((1, 1), -jnp.mean(t), jnp.float32)


def _compute_loss(dist, labels):
    d2 = dist[:N_PAIRS].reshape(1250, 128)
    l2 = labels.reshape(1250, 128)
    out = pl.pallas_call(
        _loss_body,
        out_shape=jax.ShapeDtypeStruct((1, 1), jnp.float32),
    )(d2, l2)
    return out[0, 0]


def kernel(pairs, labels, A, emb, W):
    src = pairs[:, 0]
    dst = pairs[:, 1]
    pad = jnp.zeros((B_PAD - N_PAIRS,), jnp.int32)
    src_p = jnp.concatenate([src, pad]).reshape(B_PAD // CH, CH)
    dst_p = jnp.concatenate([dst, pad]).reshape(B_PAD // CH, CH)

    x = _compute_x(emb, W[0])
    h_packed = _compute_h(A, x)
    dist = _compute_dist(src_p, dst_p, h_packed)
    return _compute_loss(dist, labels)
